# Initial kernel scaffold; baseline (speedup 1.0000x reference)
import jax
import jax.numpy as jnp
from jax.experimental import pallas as pl


def kernel(x, t1, b1, t2, b2, fc1_w, fc1_b, fc2_w, fc2_b, fc3_w, fc3_b):
    raise NotImplementedError("write your pallas kernel here")


# speedup vs baseline: 1.4293x; 1.4293x over previous
import functools

import jax
import jax.numpy as jnp
from jax.experimental import pallas as pl
from jax.experimental.pallas import tpu as pltpu

POOL = 2
VMEM_LIMIT = 48 * 1024 * 1024
G_CONV1 = 2
G_CONV2 = 4


def _sig(x):
    return pl.reciprocal(1.0 + jnp.exp(-x), approx=False)


def _conv_body(G, hp, kh, N, x_ref, t_ref, b_ref, o_ref):
    nb = kh + POOL - 1
    bands = []
    for s in range(nb):
        rows = [x_ref[g, s % 2, s // 2: s // 2 + hp] for g in range(G)]
        bands.append(rows[0] if G == 1 else jnp.concatenate(rows, axis=0))

    acc0 = None
    acc1 = None
    for i in range(kh):
        w = t_ref[i]
        d0 = jnp.dot(bands[i], w, preferred_element_type=jnp.float32)
        acc0 = d0 if acc0 is None else acc0 + d0
        d1 = jnp.dot(bands[i + 1], w, preferred_element_type=jnp.float32)
        acc1 = d1 if acc1 is None else acc1 + d1

    m = jnp.maximum(acc0, acc1)
    m = jnp.maximum(m[:, :N], m[:, N:])
    o_ref[...] = _sig(m + b_ref[...]).astype(o_ref.dtype).reshape(G, hp, N)


def _conv_stage(x_ph, t_cat, b_row, G):
    B, _, Hh, WC = x_ph.shape
    kh, _, N2 = t_cat.shape
    N = N2 // 2
    hp = Hh - kh // 2
    return pl.pallas_call(
        functools.partial(_conv_body, G, hp, kh, N),
        out_shape=jax.ShapeDtypeStruct((B, hp, N), jnp.bfloat16),
        grid=(B // G,),
        in_specs=[
            pl.BlockSpec((G, 2, Hh, WC), lambda i: (i, 0, 0, 0)),
            pl.BlockSpec((kh, WC, N2), lambda i: (0, 0, 0)),
            pl.BlockSpec((1, N), lambda i: (0, 0)),
        ],
        out_specs=pl.BlockSpec((G, hp, N), lambda i: (i, 0, 0)),
        compiler_params=pltpu.CompilerParams(
            dimension_semantics=("parallel",),
            vmem_limit_bytes=VMEM_LIMIT),
    )(x_ph, t_cat, b_row)


def _fc_body(x_ref, w1_ref, b1_ref, w2_ref, b2_ref, w3_ref, b3_ref, o_ref):
    h1 = _sig(jnp.dot(x_ref[...], w1_ref[...],
                      preferred_element_type=jnp.float32) + b1_ref[...])
    h2 = _sig(jnp.dot(h1, w2_ref[...],
                      preferred_element_type=jnp.float32) + b2_ref[...])
    o_ref[...] = (jnp.dot(h2, w3_ref[...],
                          preferred_element_type=jnp.float32) + b3_ref[...])


def _fc_stage(feat, w1, b1, w2, b2, w3, b3):
    MB, K = feat.shape
    H1, H2, NC = w1.shape[1], w2.shape[1], w3.shape[1]
    MT = MB // 2 if MB % 16 == 0 else MB
    return pl.pallas_call(
        _fc_body,
        out_shape=jax.ShapeDtypeStruct((MB, NC), jnp.float32),
        grid=(MB // MT,),
        in_specs=[
            pl.BlockSpec((MT, K), lambda i: (i, 0)),
            pl.BlockSpec((K, H1), lambda i: (0, 0)),
            pl.BlockSpec((1, H1), lambda i: (0, 0)),
            pl.BlockSpec((H1, H2), lambda i: (0, 0)),
            pl.BlockSpec((1, H2), lambda i: (0, 0)),
            pl.BlockSpec((H2, NC), lambda i: (0, 0)),
            pl.BlockSpec((1, NC), lambda i: (0, 0)),
        ],
        out_specs=pl.BlockSpec((MT, NC), lambda i: (i, 0)),
        compiler_params=pltpu.CompilerParams(
            dimension_semantics=("parallel",),
            vmem_limit_bytes=VMEM_LIMIT),
    )(feat, w1, b1.reshape(1, H1), w2, b2.reshape(1, H2), w3, b3.reshape(1, NC))


def _phase_rows(x_rows):
    return jnp.stack([x_rows[:, 0::2], x_rows[:, 1::2]], axis=1)


def _cat_phases(t):
    kh, WC, N = t.shape[1], t.shape[2], t.shape[3]
    return jnp.transpose(t, (1, 2, 0, 3)).reshape(kh, WC, 2 * N).astype(
        jnp.bfloat16)


def kernel(x, t1, b1, t2, b2, fc1_w, fc1_b, fc2_w, fc2_b, fc3_w, fc3_b):
    B, C, H, W = x.shape
    xr = jnp.transpose(x, (0, 2, 3, 1)).reshape(B, H, W * C)
    xp = _phase_rows(xr).astype(jnp.bfloat16)

    y1 = _conv_stage(xp, _cat_phases(t1), b1, G_CONV1)
    y2 = _conv_stage(_phase_rows(y1), _cat_phases(t2), b2,
                     G_CONV2)

    feat = y2.reshape(B, y2.shape[1] * y2.shape[2])
    return _fc_stage(feat, fc1_w.astype(jnp.bfloat16), fc1_b,
                     fc2_w, fc2_b, fc3_w, fc3_b)
